# Initial kernel scaffold; baseline (speedup 1.0000x reference)
import jax
import jax.numpy as jnp
from jax.experimental import pallas as pl


def kernel(v, e, edge_index, e_hull, edge_index_hull, W1, b1, W2, b2, W1h, b1h, W2h, b2h, Wc, bc):
    raise NotImplementedError("write your pallas kernel here")


# speedup vs baseline: 3.9458x; 3.9458x over previous
import functools

import jax
import jax.numpy as jnp
import numpy as np
from jax import lax
from jax.experimental import pallas as pl
from jax.experimental.pallas import tpu as pltpu
from jax.experimental.pallas import tpu_sc as plsc

def _worker_index_rows(idx2d, q, r, preload):
    w = jnp.arange(NW, dtype=jnp.int32)
    base = w * q + jnp.minimum(w, r)
    rows = base[:, None] + jnp.arange(preload, dtype=jnp.int32)[None, :]
    rows = jnp.minimum(rows, idx2d.shape[0] - 1)
    return idx2d[rows.reshape(-1)]


N = 10000
E = 320000
EH = 160000
H = 128
L = 128
NC = 2
NS = 16
NW = NC * NS
ROWS_PER_TILE = N // NS

GE = E // L
GH = EH // L
QE, RE = divmod(GE, NW)
QH, RH = divmod(GH, NW)
PRELOAD_E = 80
PRELOAD_H = 40
ROWS_A = 624
ROWS_TAIL = N - NS * ROWS_A

SHIFT = float(np.log(2.0))

_sc_mesh = plsc.VectorSubcoreMesh(core_axis_name="c", subcore_axis_name="s")


@functools.partial(
    pl.kernel,
    out_type=(
        jax.ShapeDtypeStruct((NC, N, H), jnp.float32),
        jax.ShapeDtypeStruct((NC, N, H), jnp.float32),
    ),
    mesh=_sc_mesh,
    scratch_types=[
        pltpu.VMEM_SHARED((N, H), jnp.float32),
        pltpu.VMEM((L, H), jnp.float32),
        pltpu.VMEM((PRELOAD_E, L), jnp.int32),
    ],
)
def _sc_segment_sums(e_hbm, ie_hbm, eh_hbm, ih_hbm, zeros_hbm,
                     pe_out, ph_out, accum_sh, data_v, idx_v):
    c = lax.axis_index("c")
    s = lax.axis_index("s")
    wid = s * NC + c
    myrow = s * ROWS_A
    last = s == NS - 1

    def phase(src_hbm, idx_hbm, q, r, preload, out_ref):
        pltpu.sync_copy(zeros_hbm, accum_sh.at[pl.ds(myrow, ROWS_A)])

        @pl.when(last)
        def _():
            pltpu.sync_copy(zeros_hbm.at[pl.ds(0, ROWS_TAIL)],
                            accum_sh.at[pl.ds(NS * ROWS_A, ROWS_TAIL)])

        gbase = wid * q + jnp.minimum(wid, r)
        cnt = q + (wid < r).astype(jnp.int32)
        pltpu.sync_copy(idx_hbm.at[pl.ds(wid * preload, preload)],
                        idx_v.at[pl.ds(0, preload)])
        plsc.subcore_barrier()

        def body(g, carry):
            pltpu.sync_copy(src_hbm.at[pl.ds((gbase + g) * L, L)], data_v)
            pltpu.sync_copy(data_v, accum_sh.at[idx_v.at[g]], add=True)
            return carry

        lax.fori_loop(0, cnt, body, 0)
        plsc.subcore_barrier()
        pltpu.sync_copy(accum_sh.at[pl.ds(myrow, ROWS_A)],
                        out_ref.at[c, pl.ds(myrow, ROWS_A)])

        @pl.when(last)
        def _():
            pltpu.sync_copy(accum_sh.at[pl.ds(NS * ROWS_A, ROWS_TAIL)],
                            out_ref.at[c, pl.ds(NS * ROWS_A, ROWS_TAIL)])

        plsc.subcore_barrier()

    phase(e_hbm, ie_hbm, QE, RE, PRELOAD_E, pe_out)
    phase(eh_hbm, ih_hbm, QH, RH, PRELOAD_H, ph_out)


def _act(x):
    return jnp.maximum(x, 0.0) + jnp.log1p(jnp.exp(-jnp.abs(x))) - SHIFT


def _mlp_body(pe_ref, ph_ref, v_ref, w1_ref, b1_ref, w2_ref, b2_ref,
              w1h_ref, b1h_ref, w2h_ref, b2h_ref, wc1_ref, wc2_ref, bc_ref,
              out_ref):
    f32 = jnp.float32
    a = pe_ref[0] + pe_ref[1]
    ah = ph_ref[0] + ph_ref[1]
    h = _act(jnp.dot(a, w1_ref[...], preferred_element_type=f32) + b1_ref[...])
    h = jnp.dot(h, w2_ref[...], preferred_element_type=f32) + b2_ref[...]
    hh = _act(jnp.dot(ah, w1h_ref[...], preferred_element_type=f32) + b1h_ref[...])
    hh = jnp.dot(hh, w2h_ref[...], preferred_element_type=f32) + b2h_ref[...]
    o = _act(jnp.dot(h, wc1_ref[...], preferred_element_type=f32)
             + jnp.dot(hh, wc2_ref[...], preferred_element_type=f32)
             + bc_ref[...])
    out_ref[...] = v_ref[...] + o


_MLP_B = 1000


def _mlp_call(pe, ph, v, *weights):
    part_spec = pl.BlockSpec((NC, _MLP_B, H), lambda i: (0, i, 0))
    row_spec = pl.BlockSpec((_MLP_B, H), lambda i: (i, 0))
    w_spec = pl.BlockSpec((H, H), lambda i: (0, 0))
    b_spec = pl.BlockSpec((1, H), lambda i: (0, 0))
    w_specs = [w_spec, b_spec] * 4 + [w_spec, w_spec, b_spec]
    return pl.pallas_call(
        _mlp_body,
        grid=(N // _MLP_B,),
        in_specs=[part_spec, part_spec, row_spec] + w_specs,
        out_specs=row_spec,
        out_shape=jax.ShapeDtypeStruct((N, H), jnp.float32),
    )(pe, ph, v, *weights)


def kernel(v, e, edge_index, e_hull, edge_index_hull,
           W1, b1, W2, b2, W1h, b1h, W2h, b2h, Wc, bc):
    ie = _worker_index_rows(edge_index[1].reshape(GE, L), QE, RE, PRELOAD_E)
    ih = _worker_index_rows(edge_index_hull[1].reshape(GH, L), QH, RH, PRELOAD_H)
    zeros = jnp.zeros((ROWS_A, H), jnp.float32)
    pe, ph = _sc_segment_sums(e, ie, e_hull, ih, zeros)
    return _mlp_call(
        pe, ph, v,
        W1.T, b1[None], W2.T, b2[None],
        W1h.T, b1h[None], W2h.T, b2h[None],
        Wc[:, :H].T, Wc[:, H:].T, bc[None])


# speedup vs baseline: 4.9862x; 1.2637x over previous
import functools

import jax
import jax.numpy as jnp
import numpy as np
from jax import lax
from jax.experimental import pallas as pl
from jax.experimental.pallas import tpu as pltpu
from jax.experimental.pallas import tpu_sc as plsc

def _worker_index_rows(idx2d, q, r, preload):
    w = jnp.arange(NW, dtype=jnp.int32)
    base = w * q + jnp.minimum(w, r)
    rows = base[:, None] + jnp.arange(preload, dtype=jnp.int32)[None, :]
    rows = jnp.minimum(rows, idx2d.shape[0] - 1)
    return idx2d[rows.reshape(-1)]


N = 10000
E = 320000
EH = 160000
H = 128
L = 128
NC = 2
NS = 16
NW = NC * NS
ROWS_PER_TILE = N // NS

GE = E // L
GH = EH // L
QE, RE = divmod(GE, NW)
QH, RH = divmod(GH, NW)
PRELOAD_E = 80
PRELOAD_H = 40
ROWS_A = 624
ROWS_TAIL = N - NS * ROWS_A

SHIFT = float(np.log(2.0))

_sc_mesh = plsc.VectorSubcoreMesh(core_axis_name="c", subcore_axis_name="s")


@functools.partial(
    pl.kernel,
    out_type=(
        jax.ShapeDtypeStruct((NC, N, H), jnp.float32),
        jax.ShapeDtypeStruct((NC, N, H), jnp.float32),
    ),
    mesh=_sc_mesh,
    scratch_types=[
        pltpu.VMEM_SHARED((N, H), jnp.float32),
        pltpu.VMEM((2, L, H), jnp.float32),
        pltpu.VMEM((PRELOAD_E, L), jnp.int32),
        pltpu.SemaphoreType.DMA((2,)),
        pltpu.SemaphoreType.DMA((2,)),
    ],
)
def _sc_segment_sums(e_hbm, ie_hbm, eh_hbm, ih_hbm, zeros_hbm,
                     pe_out, ph_out, accum_sh, data_v, idx_v, sem_g, sem_s):
    c = lax.axis_index("c")
    s = lax.axis_index("s")
    wid = s * NC + c
    myrow = s * ROWS_A
    last = s == NS - 1

    def phase(src_hbm, idx_hbm, q, r, preload, out_ref):
        pltpu.sync_copy(zeros_hbm, accum_sh.at[pl.ds(myrow, ROWS_A)])

        @pl.when(last)
        def _():
            pltpu.sync_copy(zeros_hbm.at[pl.ds(0, ROWS_TAIL)],
                            accum_sh.at[pl.ds(NS * ROWS_A, ROWS_TAIL)])

        gbase = wid * q + jnp.minimum(wid, r)
        cnt = q + (wid < r).astype(jnp.int32)
        pltpu.sync_copy(idx_hbm.at[pl.ds(wid * preload, preload)],
                        idx_v.at[pl.ds(0, preload)])
        plsc.subcore_barrier()

        NBUF, K = 2, 1

        def wait_gather(b):
            pltpu.make_async_copy(src_hbm.at[pl.ds(0, L)], data_v.at[b],
                                  sem_g.at[b]).wait()

        def wait_scatter(b):
            pltpu.make_async_copy(data_v.at[b],
                                  accum_sh.at[idx_v.at[0]],
                                  sem_s.at[b]).wait()

        def start_gather(g, b):
            pltpu.async_copy(src_hbm.at[pl.ds((gbase + g) * L, L)],
                             data_v.at[b], sem_g.at[b])

        for b in range(K):
            start_gather(b, b)

        def body(i, carry):
            b = lax.rem(i, NBUF)
            wait_gather(b)
            pltpu.async_copy(data_v.at[b], accum_sh.at[idx_v.at[i]],
                             sem_s.at[b], add=True)
            p = i + K
            bp = lax.rem(p, NBUF)

            @pl.when(p < cnt)
            def _():
                @pl.when(p >= NBUF)
                def _():
                    wait_scatter(bp)
                start_gather(p, bp)

            return carry

        lax.fori_loop(0, cnt, body, 0)
        for b in range(NBUF):
            wait_scatter(b)
        plsc.subcore_barrier()
        pltpu.sync_copy(accum_sh.at[pl.ds(myrow, ROWS_A)],
                        out_ref.at[c, pl.ds(myrow, ROWS_A)])

        @pl.when(last)
        def _():
            pltpu.sync_copy(accum_sh.at[pl.ds(NS * ROWS_A, ROWS_TAIL)],
                            out_ref.at[c, pl.ds(NS * ROWS_A, ROWS_TAIL)])

        plsc.subcore_barrier()

    phase(e_hbm, ie_hbm, QE, RE, PRELOAD_E, pe_out)
    phase(eh_hbm, ih_hbm, QH, RH, PRELOAD_H, ph_out)


def _act(x):
    return jnp.maximum(x, 0.0) + jnp.log1p(jnp.exp(-jnp.abs(x))) - SHIFT


def _mlp_body(pe_ref, ph_ref, v_ref, w1_ref, b1_ref, w2_ref, b2_ref,
              w1h_ref, b1h_ref, w2h_ref, b2h_ref, wc1_ref, wc2_ref, bc_ref,
              out_ref):
    f32 = jnp.float32
    a = pe_ref[0] + pe_ref[1]
    ah = ph_ref[0] + ph_ref[1]
    h = _act(jnp.dot(a, w1_ref[...], preferred_element_type=f32) + b1_ref[...])
    h = jnp.dot(h, w2_ref[...], preferred_element_type=f32) + b2_ref[...]
    hh = _act(jnp.dot(ah, w1h_ref[...], preferred_element_type=f32) + b1h_ref[...])
    hh = jnp.dot(hh, w2h_ref[...], preferred_element_type=f32) + b2h_ref[...]
    o = _act(jnp.dot(h, wc1_ref[...], preferred_element_type=f32)
             + jnp.dot(hh, wc2_ref[...], preferred_element_type=f32)
             + bc_ref[...])
    out_ref[...] = v_ref[...] + o


_MLP_B = 1000


def _mlp_call(pe, ph, v, *weights):
    part_spec = pl.BlockSpec((NC, _MLP_B, H), lambda i: (0, i, 0))
    row_spec = pl.BlockSpec((_MLP_B, H), lambda i: (i, 0))
    w_spec = pl.BlockSpec((H, H), lambda i: (0, 0))
    b_spec = pl.BlockSpec((1, H), lambda i: (0, 0))
    w_specs = [w_spec, b_spec] * 4 + [w_spec, w_spec, b_spec]
    return pl.pallas_call(
        _mlp_body,
        grid=(N // _MLP_B,),
        in_specs=[part_spec, part_spec, row_spec] + w_specs,
        out_specs=row_spec,
        out_shape=jax.ShapeDtypeStruct((N, H), jnp.float32),
    )(pe, ph, v, *weights)


def kernel(v, e, edge_index, e_hull, edge_index_hull,
           W1, b1, W2, b2, W1h, b1h, W2h, b2h, Wc, bc):
    ie = _worker_index_rows(edge_index[1].reshape(GE, L), QE, RE, PRELOAD_E)
    ih = _worker_index_rows(edge_index_hull[1].reshape(GH, L), QH, RH, PRELOAD_H)
    zeros = jnp.zeros((ROWS_A, H), jnp.float32)
    pe, ph = _sc_segment_sums(e, ie, e_hull, ih, zeros)
    return _mlp_call(
        pe, ph, v,
        W1.T, b1[None], W2.T, b2[None],
        W1h.T, b1h[None], W2h.T, b2h[None],
        Wc[:, :H].T, Wc[:, H:].T, bc[None])


# speedup vs baseline: 5.4667x; 1.0964x over previous
import functools

import jax
import jax.numpy as jnp
import numpy as np
from jax import lax
from jax.experimental import pallas as pl
from jax.experimental.pallas import tpu as pltpu
from jax.experimental.pallas import tpu_sc as plsc

def _worker_index_rows(idx2d, q, r, preload):
    w = jnp.arange(NW, dtype=jnp.int32)
    base = w * q + jnp.minimum(w, r)
    rows = base[:, None] + jnp.arange(preload, dtype=jnp.int32)[None, :]
    rows = jnp.minimum(rows, idx2d.shape[0] - 1)
    return idx2d[rows.reshape(-1)]


N = 10000
E = 320000
EH = 160000
H = 128
L = 128
NC = 2
NS = 16
NW = NC * NS
ROWS_PER_TILE = N // NS

GE = E // L
GH = EH // L
QE, RE = divmod(GE, NW)
QH, RH = divmod(GH, NW)
PRELOAD_E = 80
PRELOAD_H = 40
ROWS_A = 624
ROWS_TAIL = N - NS * ROWS_A

SHIFT = float(np.log(2.0))

_sc_mesh = plsc.VectorSubcoreMesh(core_axis_name="c", subcore_axis_name="s")


@functools.partial(
    pl.kernel,
    out_type=(
        jax.ShapeDtypeStruct((NC, N, H), jnp.float32),
        jax.ShapeDtypeStruct((NC, N, H), jnp.float32),
    ),
    mesh=_sc_mesh,
    scratch_types=[
        pltpu.VMEM_SHARED((N, H), jnp.float32),
        pltpu.VMEM((3, L, H), jnp.float32),
        pltpu.VMEM((8, L), jnp.int32),
        pltpu.SemaphoreType.DMA((3,)),
        pltpu.SemaphoreType.DMA((3,)),
    ],
)
def _sc_segment_sums(e_hbm, ie_hbm, eh_hbm, ih_hbm, zeros_hbm,
                     pe_out, ph_out, accum_sh, data_v, idx_v, sem_g, sem_s):
    c = lax.axis_index("c")
    s = lax.axis_index("s")
    wid = s * NC + c
    myrow = s * ROWS_A
    last = s == NS - 1

    def phase(src_hbm, idx_hbm, q, r, preload, out_ref):
        pltpu.sync_copy(zeros_hbm, accum_sh.at[pl.ds(myrow, ROWS_A)])

        @pl.when(last)
        def _():
            pltpu.sync_copy(zeros_hbm.at[pl.ds(0, ROWS_TAIL)],
                            accum_sh.at[pl.ds(NS * ROWS_A, ROWS_TAIL)])

        gbase = wid * q + jnp.minimum(wid, r)
        cnt = q + (wid < r).astype(jnp.int32)
        plsc.subcore_barrier()

        NBUF, D, IB = 3, 1, 8

        def wait_gather(b):
            pltpu.make_async_copy(src_hbm.at[pl.ds(0, L)], data_v.at[b],
                                  sem_g.at[b]).wait()

        def wait_scatter(b):
            pltpu.make_async_copy(data_v.at[b],
                                  accum_sh.at[idx_v.at[0]],
                                  sem_s.at[b]).wait()

        def start_gather(g, b):
            pltpu.async_copy(src_hbm.at[pl.ds((gbase + g) * L, L)],
                             data_v.at[b], sem_g.at[b])

        for g in range(D):
            start_gather(g, g % NBUF)

        def block(blk, carry):
            pltpu.sync_copy(idx_hbm.at[pl.ds(wid * preload + blk * IB, IB)],
                            idx_v)
            for k in range(IB):
                sj = blk * IB + k
                gi = sj + D
                b = lax.rem(gi, NBUF)

                @pl.when(gi < cnt)
                def _():
                    @pl.when(gi >= NBUF)
                    def _():
                        wait_scatter(b)
                    start_gather(gi, b)

                bj = lax.rem(sj, NBUF)

                @pl.when(sj < cnt)
                def _():
                    wait_gather(bj)
                    pltpu.async_copy(data_v.at[bj],
                                     accum_sh.at[idx_v.at[k]],
                                     sem_s.at[bj], add=True)

            return carry

        lax.fori_loop(0, (cnt + IB - 1) // IB, block, 0)
        for b in range(NBUF):
            wait_scatter(b)
        plsc.subcore_barrier()
        pltpu.sync_copy(accum_sh.at[pl.ds(myrow, ROWS_A)],
                        out_ref.at[c, pl.ds(myrow, ROWS_A)])

        @pl.when(last)
        def _():
            pltpu.sync_copy(accum_sh.at[pl.ds(NS * ROWS_A, ROWS_TAIL)],
                            out_ref.at[c, pl.ds(NS * ROWS_A, ROWS_TAIL)])

        plsc.subcore_barrier()

    phase(e_hbm, ie_hbm, QE, RE, PRELOAD_E, pe_out)
    phase(eh_hbm, ih_hbm, QH, RH, PRELOAD_H, ph_out)


def _act(x):
    return jnp.maximum(x, 0.0) + jnp.log1p(jnp.exp(-jnp.abs(x))) - SHIFT


def _mlp_body(pe_ref, ph_ref, v_ref, w1_ref, b1_ref, w2_ref, b2_ref,
              w1h_ref, b1h_ref, w2h_ref, b2h_ref, wc1_ref, wc2_ref, bc_ref,
              out_ref):
    f32 = jnp.float32
    a = pe_ref[0] + pe_ref[1]
    ah = ph_ref[0] + ph_ref[1]
    h = _act(jnp.dot(a, w1_ref[...], preferred_element_type=f32) + b1_ref[...])
    h = jnp.dot(h, w2_ref[...], preferred_element_type=f32) + b2_ref[...]
    hh = _act(jnp.dot(ah, w1h_ref[...], preferred_element_type=f32) + b1h_ref[...])
    hh = jnp.dot(hh, w2h_ref[...], preferred_element_type=f32) + b2h_ref[...]
    o = _act(jnp.dot(h, wc1_ref[...], preferred_element_type=f32)
             + jnp.dot(hh, wc2_ref[...], preferred_element_type=f32)
             + bc_ref[...])
    out_ref[...] = v_ref[...] + o


_MLP_B = 1000


def _mlp_call(pe, ph, v, *weights):
    part_spec = pl.BlockSpec((NC, _MLP_B, H), lambda i: (0, i, 0))
    row_spec = pl.BlockSpec((_MLP_B, H), lambda i: (i, 0))
    w_spec = pl.BlockSpec((H, H), lambda i: (0, 0))
    b_spec = pl.BlockSpec((1, H), lambda i: (0, 0))
    w_specs = [w_spec, b_spec] * 4 + [w_spec, w_spec, b_spec]
    return pl.pallas_call(
        _mlp_body,
        grid=(N // _MLP_B,),
        in_specs=[part_spec, part_spec, row_spec] + w_specs,
        out_specs=row_spec,
        out_shape=jax.ShapeDtypeStruct((N, H), jnp.float32),
    )(pe, ph, v, *weights)


def kernel(v, e, edge_index, e_hull, edge_index_hull,
           W1, b1, W2, b2, W1h, b1h, W2h, b2h, Wc, bc):
    ie = _worker_index_rows(edge_index[1].reshape(GE, L), QE, RE, PRELOAD_E)
    ih = _worker_index_rows(edge_index_hull[1].reshape(GH, L), QH, RH, PRELOAD_H)
    zeros = jnp.zeros((ROWS_A, H), jnp.float32)
    pe, ph = _sc_segment_sums(e, ie, e_hull, ih, zeros)
    return _mlp_call(
        pe, ph, v,
        W1.T, b1[None], W2.T, b2[None],
        W1h.T, b1h[None], W2h.T, b2h[None],
        Wc[:, :H].T, Wc[:, H:].T, bc[None])


# speedup vs baseline: 5.4677x; 1.0002x over previous
import functools

import jax
import jax.numpy as jnp
import numpy as np
from jax import lax
from jax.experimental import pallas as pl
from jax.experimental.pallas import tpu as pltpu
from jax.experimental.pallas import tpu_sc as plsc

def _worker_index_rows(idx2d, q, r, preload):
    w = jnp.arange(NW, dtype=jnp.int32)
    base = w * q + jnp.minimum(w, r)
    rows = base[:, None] + jnp.arange(preload, dtype=jnp.int32)[None, :]
    rows = jnp.minimum(rows, idx2d.shape[0] - 1)
    return idx2d[rows.reshape(-1)]


N = 10000
E = 320000
EH = 160000
H = 128
L = 128
NC = 2
NS = 16
NW = NC * NS
ROWS_PER_TILE = N // NS

GE = E // L
GH = EH // L
QE, RE = divmod(GE, NW)
QH, RH = divmod(GH, NW)
PRELOAD_E = 80
PRELOAD_H = 40
ROWS_A = 624
ROWS_TAIL = N - NS * ROWS_A

SHIFT = float(np.log(2.0))

_sc_mesh = plsc.VectorSubcoreMesh(core_axis_name="c", subcore_axis_name="s")


@functools.partial(
    pl.kernel,
    out_type=(
        jax.ShapeDtypeStruct((NC, N, H), jnp.float32),
        jax.ShapeDtypeStruct((NC, N, H), jnp.float32),
    ),
    mesh=_sc_mesh,
    scratch_types=[
        pltpu.VMEM_SHARED((N, H), jnp.float32),
        pltpu.VMEM((3, L, H), jnp.float32),
        pltpu.VMEM((8, L), jnp.int32),
        pltpu.SemaphoreType.DMA((3,)),
        pltpu.SemaphoreType.DMA((3,)),
    ],
)
def _sc_segment_sums(e_hbm, ie_hbm, eh_hbm, ih_hbm, zeros_hbm,
                     pe_out, ph_out, accum_sh, data_v, idx_v, sem_g, sem_s):
    c = lax.axis_index("c")
    s = lax.axis_index("s")
    wid = s * NC + c
    myrow = s * ROWS_A
    last = s == NS - 1

    def phase(src_hbm, idx_hbm, q, r, preload, out_ref):
        pltpu.sync_copy(zeros_hbm, accum_sh.at[pl.ds(myrow, ROWS_A)])

        @pl.when(last)
        def _():
            pltpu.sync_copy(zeros_hbm.at[pl.ds(0, ROWS_TAIL)],
                            accum_sh.at[pl.ds(NS * ROWS_A, ROWS_TAIL)])

        gbase = wid * q + jnp.minimum(wid, r)
        cnt = q + (wid < r).astype(jnp.int32)
        plsc.subcore_barrier()

        NBUF, D, IB = 3, 1, 8

        def wait_gather(b):
            pltpu.make_async_copy(src_hbm.at[pl.ds(0, L)], data_v.at[b],
                                  sem_g.at[b]).wait()

        def wait_scatter(b):
            pltpu.make_async_copy(data_v.at[b],
                                  accum_sh.at[idx_v.at[0]],
                                  sem_s.at[b]).wait()

        def start_gather(g, b):
            pltpu.async_copy(src_hbm.at[pl.ds((gbase + g) * L, L)],
                             data_v.at[b], sem_g.at[b])

        for g in range(D):
            start_gather(g, g % NBUF)

        def block(blk, carry):
            pltpu.sync_copy(idx_hbm.at[pl.ds(wid * preload + blk * IB, IB)],
                            idx_v)
            for k in range(IB):
                sj = blk * IB + k
                gi = sj + D
                b = lax.rem(gi, NBUF)

                @pl.when(gi < cnt)
                def _():
                    @pl.when(gi >= NBUF)
                    def _():
                        wait_scatter(b)
                    start_gather(gi, b)

                bj = lax.rem(sj, NBUF)

                @pl.when(sj < cnt)
                def _():
                    wait_gather(bj)
                    pltpu.async_copy(data_v.at[bj],
                                     accum_sh.at[idx_v.at[k]],
                                     sem_s.at[bj], add=True)

            return carry

        lax.fori_loop(0, (cnt + IB - 1) // IB, block, 0)
        for b in range(NBUF):
            wait_scatter(b)
        plsc.subcore_barrier()
        pltpu.sync_copy(accum_sh.at[pl.ds(myrow, ROWS_A)],
                        out_ref.at[c, pl.ds(myrow, ROWS_A)])

        @pl.when(last)
        def _():
            pltpu.sync_copy(accum_sh.at[pl.ds(NS * ROWS_A, ROWS_TAIL)],
                            out_ref.at[c, pl.ds(NS * ROWS_A, ROWS_TAIL)])

        plsc.subcore_barrier()

    phase(e_hbm, ie_hbm, QE, RE, PRELOAD_E, pe_out)
    phase(eh_hbm, ih_hbm, QH, RH, PRELOAD_H, ph_out)


def _act(x):
    return jnp.maximum(x, 0.0) + jnp.log1p(jnp.exp(-jnp.abs(x))) - SHIFT


def _matT(x, w_ref):
    return lax.dot_general(x, w_ref[...], (((1,), (1,)), ((), ())),
                           preferred_element_type=jnp.float32)


def _mlp_body(pe_ref, ph_ref, v_ref, w1_ref, b1_ref, w2_ref, b2_ref,
              w1h_ref, b1h_ref, w2h_ref, b2h_ref, wc1_ref, wc2_ref, bc_ref,
              out_ref):
    a = pe_ref[0] + pe_ref[1]
    ah = ph_ref[0] + ph_ref[1]
    h = _matT(_act(_matT(a, w1_ref) + b1_ref[...]), w2_ref) + b2_ref[...]
    hh = _matT(_act(_matT(ah, w1h_ref) + b1h_ref[...]), w2h_ref) + b2h_ref[...]
    o = _act(_matT(h, wc1_ref) + _matT(hh, wc2_ref) + bc_ref[...])
    out_ref[...] = v_ref[...] + o


_MLP_B = 1000


def _mlp_call(pe, ph, v, *weights):
    part_spec = pl.BlockSpec((NC, _MLP_B, H), lambda i: (0, i, 0))
    row_spec = pl.BlockSpec((_MLP_B, H), lambda i: (i, 0))
    w_spec = pl.BlockSpec((H, H), lambda i: (0, 0))
    b_spec = pl.BlockSpec((1, H), lambda i: (0, 0))
    w_specs = [w_spec, b_spec] * 4 + [w_spec, w_spec, b_spec]
    return pl.pallas_call(
        _mlp_body,
        grid=(N // _MLP_B,),
        in_specs=[part_spec, part_spec, row_spec] + w_specs,
        out_specs=row_spec,
        out_shape=jax.ShapeDtypeStruct((N, H), jnp.float32),
    )(pe, ph, v, *weights)


def kernel(v, e, edge_index, e_hull, edge_index_hull,
           W1, b1, W2, b2, W1h, b1h, W2h, b2h, Wc, bc):
    ie = _worker_index_rows(edge_index[1].reshape(GE, L), QE, RE, PRELOAD_E)
    ih = _worker_index_rows(edge_index_hull[1].reshape(GH, L), QH, RH, PRELOAD_H)
    zeros = jnp.zeros((ROWS_A, H), jnp.float32)
    pe, ph = _sc_segment_sums(e, ie, e_hull, ih, zeros)
    return _mlp_call(
        pe, ph, v,
        W1, b1[None], W2, b2[None],
        W1h, b1h[None], W2h, b2h[None],
        Wc[:, :H], Wc[:, H:], bc[None])


# speedup vs baseline: 5.7490x; 1.0514x over previous
import functools

import jax
import jax.numpy as jnp
import numpy as np
from jax import lax
from jax.experimental import pallas as pl
from jax.experimental.pallas import tpu as pltpu
from jax.experimental.pallas import tpu_sc as plsc

N = 10000
E = 320000
EH = 160000
H = 128
L = 128
NC = 2
NS = 16
NW = NC * NS
IB = 8

GE = E // L
GH = EH // L
BLK_E, TAIL_E = divmod(GE, IB)
BLK_H, TAIL_H = divmod(GH, IB)
QBE, RBE = divmod(BLK_E, NW)
QBH, RBH = divmod(BLK_H, NW)
PAD_E = (BLK_E + 1) * IB
PAD_H = (BLK_H + 1) * IB

ROWS_A = 624
ROWS_TAIL = N - NS * ROWS_A

SHIFT = float(np.log(2.0))

_sc_mesh = plsc.VectorSubcoreMesh(core_axis_name="c", subcore_axis_name="s")


@functools.partial(
    pl.kernel,
    out_type=(
        jax.ShapeDtypeStruct((NC, N, H), jnp.float32),
        jax.ShapeDtypeStruct((NC, N, H), jnp.float32),
    ),
    mesh=_sc_mesh,
    scratch_types=[
        pltpu.VMEM_SHARED((N, H), jnp.float32),
        pltpu.VMEM((3, L, H), jnp.float32),
        pltpu.VMEM((IB, L), jnp.int32),
        pltpu.SemaphoreType.DMA((3,)),
        pltpu.SemaphoreType.DMA((3,)),
    ],
)
def _sc_segment_sums(e_hbm, ie_hbm, eh_hbm, ih_hbm, zeros_hbm,
                     pe_out, ph_out, accum_sh, data_v, idx_v, sem_g, sem_s):
    c = lax.axis_index("c")
    s = lax.axis_index("s")
    wid = s * NC + c
    myrow = s * ROWS_A
    last = s == NS - 1

    def phase(src_hbm, idx_hbm, qb, rb, tail, out_ref):
        pltpu.sync_copy(zeros_hbm, accum_sh.at[pl.ds(myrow, ROWS_A)])

        @pl.when(last)
        def _():
            pltpu.sync_copy(zeros_hbm.at[pl.ds(0, ROWS_TAIL)],
                            accum_sh.at[pl.ds(NS * ROWS_A, ROWS_TAIL)])

        gbase = (wid * qb + jnp.minimum(wid, rb)) * IB
        cnt = (qb + (wid < rb).astype(jnp.int32)) * IB
        cnt = cnt + jnp.where(wid == NW - 1, tail, 0)
        plsc.subcore_barrier()

        NBUF, D = 3, 1

        def wait_gather(b):
            pltpu.make_async_copy(src_hbm.at[pl.ds(0, L)], data_v.at[b],
                                  sem_g.at[b]).wait()

        def wait_scatter(b):
            pltpu.make_async_copy(data_v.at[b],
                                  accum_sh.at[idx_v.at[0]],
                                  sem_s.at[b]).wait()

        def start_gather(g, b):
            pltpu.async_copy(src_hbm.at[pl.ds((gbase + g) * L, L)],
                             data_v.at[b], sem_g.at[b])

        for g in range(D):
            start_gather(g, g % NBUF)

        def block(blk, carry):
            pltpu.sync_copy(idx_hbm.at[pl.ds(gbase + blk * IB, IB)], idx_v)
            for k in range(IB):
                sj = blk * IB + k
                gi = sj + D
                b = lax.rem(gi, NBUF)

                @pl.when(gi < cnt)
                def _():
                    @pl.when(gi >= NBUF)
                    def _():
                        wait_scatter(b)
                    start_gather(gi, b)

                bj = lax.rem(sj, NBUF)

                @pl.when(sj < cnt)
                def _():
                    wait_gather(bj)
                    pltpu.async_copy(data_v.at[bj],
                                     accum_sh.at[idx_v.at[k]],
                                     sem_s.at[bj], add=True)

            return carry

        lax.fori_loop(0, (cnt + IB - 1) // IB, block, 0)
        for b in range(NBUF):
            wait_scatter(b)
        plsc.subcore_barrier()
        pltpu.sync_copy(accum_sh.at[pl.ds(myrow, ROWS_A)],
                        out_ref.at[c, pl.ds(myrow, ROWS_A)])

        @pl.when(last)
        def _():
            pltpu.sync_copy(accum_sh.at[pl.ds(NS * ROWS_A, ROWS_TAIL)],
                            out_ref.at[c, pl.ds(NS * ROWS_A, ROWS_TAIL)])

        plsc.subcore_barrier()

    phase(e_hbm, ie_hbm, QBE, RBE, TAIL_E, pe_out)
    phase(eh_hbm, ih_hbm, QBH, RBH, TAIL_H, ph_out)


def _act(x):
    return jnp.maximum(x, 0.0) + jnp.log1p(jnp.exp(-jnp.abs(x))) - SHIFT


def _matT(x, w_ref):
    return lax.dot_general(x, w_ref[...], (((1,), (1,)), ((), ())),
                           preferred_element_type=jnp.float32)


def _mlp_body(pe_ref, ph_ref, v_ref, w1_ref, b1_ref, w2_ref, b2_ref,
              w1h_ref, b1h_ref, w2h_ref, b2h_ref, wc1_ref, wc2_ref, bc_ref,
              out_ref):
    a = pe_ref[0] + pe_ref[1]
    ah = ph_ref[0] + ph_ref[1]
    h = _matT(_act(_matT(a, w1_ref) + b1_ref[...]), w2_ref) + b2_ref[...]
    hh = _matT(_act(_matT(ah, w1h_ref) + b1h_ref[...]), w2h_ref) + b2h_ref[...]
    o = _act(_matT(h, wc1_ref) + _matT(hh, wc2_ref) + bc_ref[...])
    out_ref[...] = v_ref[...] + o


_MLP_B = 1000


def _mlp_call(pe, ph, v, *weights):
    part_spec = pl.BlockSpec((NC, _MLP_B, H), lambda i: (0, i, 0))
    row_spec = pl.BlockSpec((_MLP_B, H), lambda i: (i, 0))
    w_spec = pl.BlockSpec((H, H), lambda i: (0, 0))
    b_spec = pl.BlockSpec((1, H), lambda i: (0, 0))
    w_specs = [w_spec, b_spec] * 4 + [w_spec, w_spec, b_spec]
    return pl.pallas_call(
        _mlp_body,
        grid=(N // _MLP_B,),
        in_specs=[part_spec, part_spec, row_spec] + w_specs,
        out_specs=row_spec,
        out_shape=jax.ShapeDtypeStruct((N, H), jnp.float32),
    )(pe, ph, v, *weights)


def kernel(v, e, edge_index, e_hull, edge_index_hull,
           W1, b1, W2, b2, W1h, b1h, W2h, b2h, Wc, bc):
    ie = jnp.pad(edge_index[1].reshape(GE, L), ((0, PAD_E - GE), (0, 0)))
    ih = jnp.pad(edge_index_hull[1].reshape(GH, L), ((0, PAD_H - GH), (0, 0)))
    zeros = jnp.zeros((ROWS_A, H), jnp.float32)
    pe, ph = _sc_segment_sums(e, ie, e_hull, ih, zeros)
    return _mlp_call(
        pe, ph, v,
        W1, b1[None], W2, b2[None],
        W1h, b1h[None], W2h, b2h[None],
        Wc[:, :H], Wc[:, H:], bc[None])


# speedup vs baseline: 5.8175x; 1.0119x over previous
import functools

import jax
import jax.numpy as jnp
import numpy as np
from jax import lax
from jax.experimental import pallas as pl
from jax.experimental.pallas import tpu as pltpu
from jax.experimental.pallas import tpu_sc as plsc

N = 10000
E = 320000
EH = 160000
H = 128
L = 128
NC = 2
NS = 16
NW = NC * NS
IB = 8

GE = E // L
GH = EH // L
BLK_E, TAIL_E = divmod(GE, IB)
BLK_H, TAIL_H = divmod(GH, IB)
QBE, RBE = divmod(BLK_E, NW)
QBH, RBH = divmod(BLK_H, NW)
PAD_E = (BLK_E + 1) * IB
PAD_H = (BLK_H + 1) * IB

ROWS_A = 624
ROWS_TAIL = N - NS * ROWS_A

SHIFT = float(np.log(2.0))

_sc_mesh = plsc.VectorSubcoreMesh(core_axis_name="c", subcore_axis_name="s")


@functools.partial(
    pl.kernel,
    out_type=(
        jax.ShapeDtypeStruct((NC, N, H), jnp.float32),
        jax.ShapeDtypeStruct((NC, N, H), jnp.float32),
    ),
    mesh=_sc_mesh,
    scratch_types=[
        pltpu.VMEM_SHARED((N, H), jnp.float32),
        pltpu.VMEM((3, L, H), jnp.float32),
        pltpu.VMEM((IB, L), jnp.int32),
        pltpu.SemaphoreType.DMA((3,)),
        pltpu.SemaphoreType.DMA((3,)),
    ],
)
def _sc_segment_sums(e_hbm, ie_hbm, eh_hbm, ih_hbm, zeros_hbm,
                     pe_out, ph_out, accum_sh, data_v, idx_v, sem_g, sem_s):
    c = lax.axis_index("c")
    s = lax.axis_index("s")
    wid = s * NC + c
    myrow = s * ROWS_A
    last = s == NS - 1

    def phase(src_hbm, idx_hbm, qb, rb, tail, out_ref):
        gbase = (wid * qb + jnp.minimum(wid, rb)) * IB
        cnt = (qb + (wid < rb).astype(jnp.int32)) * IB
        cnt = cnt + jnp.where(wid == NW - 1, tail, 0)

        NBUF, D = 3, 1

        def wait_gather(b):
            pltpu.make_async_copy(src_hbm.at[pl.ds(0, L)], data_v.at[b],
                                  sem_g.at[b]).wait()

        def wait_scatter(b):
            pltpu.make_async_copy(data_v.at[b],
                                  accum_sh.at[idx_v.at[0]],
                                  sem_s.at[b]).wait()

        def start_gather(g, b):
            pltpu.async_copy(src_hbm.at[pl.ds((gbase + g) * L, L)],
                             data_v.at[b], sem_g.at[b])

        for g in range(D):
            start_gather(g, g % NBUF)

        pltpu.sync_copy(zeros_hbm, accum_sh.at[pl.ds(myrow, ROWS_A)])

        @pl.when(last)
        def _():
            pltpu.sync_copy(zeros_hbm.at[pl.ds(0, ROWS_TAIL)],
                            accum_sh.at[pl.ds(NS * ROWS_A, ROWS_TAIL)])

        plsc.subcore_barrier()

        def block(blk, carry):
            pltpu.sync_copy(idx_hbm.at[pl.ds(gbase + blk * IB, IB)], idx_v)
            for k in range(IB):
                sj = blk * IB + k
                gi = sj + D
                b = lax.rem(gi, NBUF)

                @pl.when(gi < cnt)
                def _():
                    @pl.when(gi >= NBUF)
                    def _():
                        wait_scatter(b)
                    start_gather(gi, b)

                bj = lax.rem(sj, NBUF)

                @pl.when(sj < cnt)
                def _():
                    wait_gather(bj)
                    pltpu.async_copy(data_v.at[bj],
                                     accum_sh.at[idx_v.at[k]],
                                     sem_s.at[bj], add=True)

            return carry

        lax.fori_loop(0, (cnt + IB - 1) // IB, block, 0)
        for b in range(NBUF):
            wait_scatter(b)
        plsc.subcore_barrier()
        pltpu.sync_copy(accum_sh.at[pl.ds(myrow, ROWS_A)],
                        out_ref.at[c, pl.ds(myrow, ROWS_A)])

        @pl.when(last)
        def _():
            pltpu.sync_copy(accum_sh.at[pl.ds(NS * ROWS_A, ROWS_TAIL)],
                            out_ref.at[c, pl.ds(NS * ROWS_A, ROWS_TAIL)])


    phase(e_hbm, ie_hbm, QBE, RBE, TAIL_E, pe_out)
    phase(eh_hbm, ih_hbm, QBH, RBH, TAIL_H, ph_out)


def _act(x):
    return jnp.maximum(x, 0.0) + jnp.log1p(jnp.exp(-jnp.abs(x))) - SHIFT


def _matT(x, w_ref):
    return lax.dot_general(x, w_ref[...], (((1,), (1,)), ((), ())),
                           preferred_element_type=jnp.float32)


def _mlp_body(pe_ref, ph_ref, v_ref, w1_ref, b1_ref, w2_ref, b2_ref,
              w1h_ref, b1h_ref, w2h_ref, b2h_ref, wc1_ref, wc2_ref, bc_ref,
              out_ref):
    a = pe_ref[0] + pe_ref[1]
    ah = ph_ref[0] + ph_ref[1]
    h = _matT(_act(_matT(a, w1_ref) + b1_ref[...]), w2_ref) + b2_ref[...]
    hh = _matT(_act(_matT(ah, w1h_ref) + b1h_ref[...]), w2h_ref) + b2h_ref[...]
    o = _act(_matT(h, wc1_ref) + _matT(hh, wc2_ref) + bc_ref[...])
    out_ref[...] = v_ref[...] + o


_MLP_B = 2000


def _mlp_call(pe, ph, v, *weights):
    part_spec = pl.BlockSpec((NC, _MLP_B, H), lambda i: (0, i, 0))
    row_spec = pl.BlockSpec((_MLP_B, H), lambda i: (i, 0))
    w_spec = pl.BlockSpec((H, H), lambda i: (0, 0))
    b_spec = pl.BlockSpec((1, H), lambda i: (0, 0))
    w_specs = [w_spec, b_spec] * 4 + [w_spec, w_spec, b_spec]
    return pl.pallas_call(
        _mlp_body,
        grid=(N // _MLP_B,),
        in_specs=[part_spec, part_spec, row_spec] + w_specs,
        out_specs=row_spec,
        out_shape=jax.ShapeDtypeStruct((N, H), jnp.float32),
    )(pe, ph, v, *weights)


def kernel(v, e, edge_index, e_hull, edge_index_hull,
           W1, b1, W2, b2, W1h, b1h, W2h, b2h, Wc, bc):
    ie = jnp.pad(edge_index[1].reshape(GE, L), ((0, PAD_E - GE), (0, 0)))
    ih = jnp.pad(edge_index_hull[1].reshape(GH, L), ((0, PAD_H - GH), (0, 0)))
    zeros = jnp.zeros((ROWS_A, H), jnp.float32)
    pe, ph = _sc_segment_sums(e, ie, e_hull, ih, zeros)
    return _mlp_call(
        pe, ph, v,
        W1, b1[None], W2, b2[None],
        W1h, b1h[None], W2h, b2h[None],
        Wc[:, :H], Wc[:, H:], bc[None])


# speedup vs baseline: 5.9441x; 1.0218x over previous
import functools

import jax
import jax.numpy as jnp
import numpy as np
from jax import lax
from jax.experimental import pallas as pl
from jax.experimental.pallas import tpu as pltpu
from jax.experimental.pallas import tpu_sc as plsc

N = 10000
E = 320000
EH = 160000
H = 128
L = 128
NC = 2
NS = 16
NW = NC * NS
IB = 8

GE = E // L
GH = EH // L
BLK_E, TAIL_E = divmod(GE, IB)
BLK_H, TAIL_H = divmod(GH, IB)
QBE, RBE = divmod(BLK_E, NW)
QBH, RBH = divmod(BLK_H, NW)
PAD_E = (BLK_E + 1) * IB
PAD_H = (BLK_H + 1) * IB

ROWS_A = 624
ROWS_TAIL = N - NS * ROWS_A

SHIFT = float(np.log(2.0))

_sc_mesh = plsc.VectorSubcoreMesh(core_axis_name="c", subcore_axis_name="s")


@functools.partial(
    pl.kernel,
    out_type=(
        jax.ShapeDtypeStruct((NC, N, H), jnp.float32),
        jax.ShapeDtypeStruct((NC, N, H), jnp.float32),
    ),
    mesh=_sc_mesh,
    scratch_types=[
        pltpu.VMEM_SHARED((N, H), jnp.float32),
        pltpu.VMEM((3, L, H), jnp.float32),
        pltpu.VMEM((IB, L), jnp.int32),
        pltpu.SemaphoreType.DMA((3,)),
        pltpu.SemaphoreType.DMA((3,)),
    ],
)
def _sc_segment_sums(e_hbm, ie_hbm, eh_hbm, ih_hbm, zeros_hbm,
                     pe_out, ph_out, accum_sh, data_v, idx_v, sem_g, sem_s):
    c = lax.axis_index("c")
    s = lax.axis_index("s")
    wid = s * NC + c
    myrow = s * ROWS_A
    last = s == NS - 1

    def phase(src_hbm, idx_hbm, qb, rb, tail, out_ref, zero_first):
        gbase = (wid * qb + jnp.minimum(wid, rb)) * IB
        cnt = (qb + (wid < rb).astype(jnp.int32)) * IB
        cnt = cnt + jnp.where(wid == NW - 1, tail, 0)

        NBUF, D = 3, 1

        def wait_gather(b):
            pltpu.make_async_copy(src_hbm.at[pl.ds(0, L)], data_v.at[b],
                                  sem_g.at[b]).wait()

        def wait_scatter(b):
            pltpu.make_async_copy(data_v.at[b],
                                  accum_sh.at[idx_v.at[0]],
                                  sem_s.at[b]).wait()

        def start_gather(g, b):
            pltpu.async_copy(src_hbm.at[pl.ds((gbase + g) * L, L)],
                             data_v.at[b], sem_g.at[b])

        for g in range(D):
            start_gather(g, g % NBUF)

        if zero_first:
            pltpu.sync_copy(zeros_hbm, accum_sh.at[pl.ds(myrow, ROWS_A)])

            @pl.when(last)
            def _():
                pltpu.sync_copy(zeros_hbm.at[pl.ds(0, ROWS_TAIL)],
                                accum_sh.at[pl.ds(NS * ROWS_A, ROWS_TAIL)])

        plsc.subcore_barrier()

        def block(blk, carry):
            pltpu.sync_copy(idx_hbm.at[pl.ds(gbase + blk * IB, IB)], idx_v)
            for k in range(IB):
                sj = blk * IB + k
                gi = sj + D
                b = lax.rem(gi, NBUF)

                @pl.when(gi < cnt)
                def _():
                    @pl.when(gi >= NBUF)
                    def _():
                        wait_scatter(b)
                    start_gather(gi, b)

                bj = lax.rem(sj, NBUF)

                @pl.when(sj < cnt)
                def _():
                    wait_gather(bj)
                    pltpu.async_copy(data_v.at[bj],
                                     accum_sh.at[idx_v.at[k]],
                                     sem_s.at[bj], add=True)

            return carry

        lax.fori_loop(0, (cnt + IB - 1) // IB, block, 0)
        for b in range(NBUF):
            wait_scatter(b)
        plsc.subcore_barrier()
        pltpu.sync_copy(accum_sh.at[pl.ds(myrow, ROWS_A)],
                        out_ref.at[c, pl.ds(myrow, ROWS_A)])

        @pl.when(last)
        def _():
            pltpu.sync_copy(accum_sh.at[pl.ds(NS * ROWS_A, ROWS_TAIL)],
                            out_ref.at[c, pl.ds(NS * ROWS_A, ROWS_TAIL)])

    phase(e_hbm, ie_hbm, QBE, RBE, TAIL_E, pe_out, zero_first=True)
    phase(eh_hbm, ih_hbm, QBH, RBH, TAIL_H, ph_out, zero_first=False)


def _act(x):
    return jnp.maximum(x, 0.0) + jnp.log1p(jnp.exp(-jnp.abs(x))) - SHIFT


def _matT(x, w_ref):
    return lax.dot_general(x, w_ref[...], (((1,), (1,)), ((), ())),
                           preferred_element_type=jnp.float32)


def _mlp_body(pe_ref, ph_ref, v_ref, w1_ref, b1_ref, w2_ref, b2_ref,
              w1h_ref, b1h_ref, w2h_ref, b2h_ref, wc1_ref, wc2_ref, bc_ref,
              out_ref):
    a = pe_ref[0] + pe_ref[1]
    ah = (ph_ref[0] + ph_ref[1]) - a
    h = _matT(_act(_matT(a, w1_ref) + b1_ref[...]), w2_ref) + b2_ref[...]
    hh = _matT(_act(_matT(ah, w1h_ref) + b1h_ref[...]), w2h_ref) + b2h_ref[...]
    o = _act(_matT(h, wc1_ref) + _matT(hh, wc2_ref) + bc_ref[...])
    out_ref[...] = v_ref[...] + o


_MLP_B = 2000


def _mlp_call(pe, ph, v, *weights):
    part_spec = pl.BlockSpec((NC, _MLP_B, H), lambda i: (0, i, 0))
    row_spec = pl.BlockSpec((_MLP_B, H), lambda i: (i, 0))
    w_spec = pl.BlockSpec((H, H), lambda i: (0, 0))
    b_spec = pl.BlockSpec((1, H), lambda i: (0, 0))
    w_specs = [w_spec, b_spec] * 4 + [w_spec, w_spec, b_spec]
    return pl.pallas_call(
        _mlp_body,
        grid=(N // _MLP_B,),
        in_specs=[part_spec, part_spec, row_spec] + w_specs,
        out_specs=row_spec,
        out_shape=jax.ShapeDtypeStruct((N, H), jnp.float32),
    )(pe, ph, v, *weights)


def kernel(v, e, edge_index, e_hull, edge_index_hull,
           W1, b1, W2, b2, W1h, b1h, W2h, b2h, Wc, bc):
    ie = jnp.pad(edge_index[1].reshape(GE, L), ((0, PAD_E - GE), (0, 0)))
    ih = jnp.pad(edge_index_hull[1].reshape(GH, L), ((0, PAD_H - GH), (0, 0)))
    zeros = jnp.zeros((ROWS_A, H), jnp.float32)
    pe, ph = _sc_segment_sums(e, ie, e_hull, ih, zeros)
    return _mlp_call(
        pe, ph, v,
        W1, b1[None], W2, b2[None],
        W1h, b1h[None], W2h, b2h[None],
        Wc[:, :H], Wc[:, H:], bc[None])


# speedup vs baseline: 6.8143x; 1.1464x over previous
import functools

import jax
import jax.numpy as jnp
import numpy as np
from jax import lax
from jax.experimental import pallas as pl
from jax.experimental.pallas import tpu as pltpu
from jax.experimental.pallas import tpu_sc as plsc

N = 10000
E = 320000
EH = 160000
H = 128
L = 128
NC = 2
NS = 16
NW = NC * NS
IB = 8

GE = E // L
GH = EH // L
BLK_E, TAIL_E = divmod(GE, IB)
BLK_H, TAIL_H = divmod(GH, IB)
QBE, RBE = divmod(BLK_E, NW)
QBH, RBH = divmod(BLK_H, NW)
PAD_E = (BLK_E + 1) * IB
PAD_H = (BLK_H + 1) * IB

ROWS_A = 624
ROWS_TAIL = N - NS * ROWS_A

SHIFT = float(np.log(2.0))

_sc_mesh = plsc.VectorSubcoreMesh(core_axis_name="c", subcore_axis_name="s")


@functools.partial(
    pl.kernel,
    out_type=(
        jax.ShapeDtypeStruct((NC, N, H), jnp.float32),
        jax.ShapeDtypeStruct((NC, N, H), jnp.float32),
    ),
    mesh=_sc_mesh,
    scratch_types=[
        pltpu.VMEM_SHARED((N, H), jnp.float32),
        pltpu.VMEM((3, L, H), jnp.float32),
        pltpu.VMEM((IB, L), jnp.int32),
        pltpu.SemaphoreType.DMA((3,)),
        pltpu.SemaphoreType.DMA((3,)),
    ],
)
def _sc_segment_sums(e_hbm, ie_hbm, eh_hbm, ih_hbm, zeros_hbm,
                     pe_out, ph_out, accum_sh, data_v, idx_v, sem_g, sem_s):
    c = lax.axis_index("c")
    s = lax.axis_index("s")
    wid = s * NC + c
    myrow = s * ROWS_A
    last = s == NS - 1

    def phase(src_hbm, idx_hbm, qb, rb, tail, out_ref, zero_first):
        gbase = (wid * qb + jnp.minimum(wid, rb)) * IB
        cnt = (qb + (wid < rb).astype(jnp.int32)) * IB
        cnt = cnt + jnp.where(wid == NW - 1, tail, 0)

        NBUF, D = 3, 2

        def wait_gather(b):
            pltpu.make_async_copy(src_hbm.at[pl.ds(0, L)], data_v.at[b],
                                  sem_g.at[b]).wait()

        def wait_scatter(b):
            pltpu.make_async_copy(data_v.at[b],
                                  accum_sh.at[idx_v.at[0]],
                                  sem_s.at[b]).wait()

        def start_gather(g, b):
            pltpu.async_copy(src_hbm.at[pl.ds((gbase + g) * L, L)],
                             data_v.at[b], sem_g.at[b])

        for g in range(D):
            start_gather(g, g % NBUF)

        if zero_first:
            pltpu.sync_copy(zeros_hbm, accum_sh.at[pl.ds(myrow, ROWS_A)])

            @pl.when(last)
            def _():
                pltpu.sync_copy(zeros_hbm.at[pl.ds(0, ROWS_TAIL)],
                                accum_sh.at[pl.ds(NS * ROWS_A, ROWS_TAIL)])

        plsc.subcore_barrier()

        def block(blk, carry):
            pltpu.sync_copy(idx_hbm.at[pl.ds(gbase + blk * IB, IB)], idx_v)
            for k in range(IB):
                sj = blk * IB + k
                gi = sj + D
                b = lax.rem(gi, NBUF)

                @pl.when(gi < cnt)
                def _():
                    @pl.when(gi >= NBUF)
                    def _():
                        wait_scatter(b)
                    start_gather(gi, b)

                bj = lax.rem(sj, NBUF)

                @pl.when(sj < cnt)
                def _():
                    wait_gather(bj)
                    pltpu.async_copy(data_v.at[bj],
                                     accum_sh.at[idx_v.at[k]],
                                     sem_s.at[bj], add=True)

            return carry

        lax.fori_loop(0, (cnt + IB - 1) // IB, block, 0)
        for b in range(NBUF):
            wait_scatter(b)
        plsc.subcore_barrier()
        pltpu.sync_copy(accum_sh.at[pl.ds(myrow, ROWS_A)],
                        out_ref.at[c, pl.ds(myrow, ROWS_A)])

        @pl.when(last)
        def _():
            pltpu.sync_copy(accum_sh.at[pl.ds(NS * ROWS_A, ROWS_TAIL)],
                            out_ref.at[c, pl.ds(NS * ROWS_A, ROWS_TAIL)])

    phase(e_hbm, ie_hbm, QBE, RBE, TAIL_E, pe_out, zero_first=True)
    phase(eh_hbm, ih_hbm, QBH, RBH, TAIL_H, ph_out, zero_first=False)


def _act(x):
    return jnp.maximum(x, 0.0) + jnp.log1p(jnp.exp(-jnp.abs(x))) - SHIFT


def _matT(x, w_ref):
    return lax.dot_general(x, w_ref[...], (((1,), (1,)), ((), ())),
                           preferred_element_type=jnp.float32)


def _mlp_body(pe_ref, ph_ref, v_ref, w1_ref, b1_ref, w2_ref, b2_ref,
              w1h_ref, b1h_ref, w2h_ref, b2h_ref, wc1_ref, wc2_ref, bc_ref,
              out_ref):
    a = pe_ref[0] + pe_ref[1]
    ah = (ph_ref[0] + ph_ref[1]) - a
    h = _matT(_act(_matT(a, w1_ref) + b1_ref[...]), w2_ref) + b2_ref[...]
    hh = _matT(_act(_matT(ah, w1h_ref) + b1h_ref[...]), w2h_ref) + b2h_ref[...]
    o = _act(_matT(h, wc1_ref) + _matT(hh, wc2_ref) + bc_ref[...])
    out_ref[...] = v_ref[...] + o


_MLP_B = 2000


def _mlp_call(pe, ph, v, *weights):
    part_spec = pl.BlockSpec((NC, _MLP_B, H), lambda i: (0, i, 0))
    row_spec = pl.BlockSpec((_MLP_B, H), lambda i: (i, 0))
    w_spec = pl.BlockSpec((H, H), lambda i: (0, 0))
    b_spec = pl.BlockSpec((1, H), lambda i: (0, 0))
    w_specs = [w_spec, b_spec] * 4 + [w_spec, w_spec, b_spec]
    return pl.pallas_call(
        _mlp_body,
        grid=(N // _MLP_B,),
        in_specs=[part_spec, part_spec, row_spec] + w_specs,
        out_specs=row_spec,
        out_shape=jax.ShapeDtypeStruct((N, H), jnp.float32),
    )(pe, ph, v, *weights)


def kernel(v, e, edge_index, e_hull, edge_index_hull,
           W1, b1, W2, b2, W1h, b1h, W2h, b2h, Wc, bc):
    ie = jnp.pad(edge_index[1].reshape(GE, L), ((0, PAD_E - GE), (0, 0)))
    ih = jnp.pad(edge_index_hull[1].reshape(GH, L), ((0, PAD_H - GH), (0, 0)))
    zeros = jnp.zeros((ROWS_A, H), jnp.float32)
    pe, ph = _sc_segment_sums(e, ie, e_hull, ih, zeros)
    return _mlp_call(
        pe, ph, v,
        W1, b1[None], W2, b2[None],
        W1h, b1h[None], W2h, b2h[None],
        Wc[:, :H], Wc[:, H:], bc[None])


# speedup vs baseline: 7.3955x; 1.0853x over previous
import functools

import jax
import jax.numpy as jnp
import numpy as np
from jax import lax
from jax.experimental import pallas as pl
from jax.experimental.pallas import tpu as pltpu
from jax.experimental.pallas import tpu_sc as plsc

N = 10000
E = 320000
EH = 160000
H = 128
L = 128
NC = 2
NS = 16
NW = NC * NS
IB = 8

GE = E // L
GH = EH // L
BLK_E, TAIL_E = divmod(GE, IB)
BLK_H, TAIL_H = divmod(GH, IB)
QBE, RBE = divmod(BLK_E, NW)
QBH, RBH = divmod(BLK_H, NW)
PAD_E = (BLK_E + 1) * IB
PAD_H = (BLK_H + 1) * IB

ROWS_A = 624
ROWS_TAIL = N - NS * ROWS_A

SHIFT = float(np.log(2.0))

_sc_mesh = plsc.VectorSubcoreMesh(core_axis_name="c", subcore_axis_name="s")


@functools.partial(
    pl.kernel,
    out_type=(
        jax.ShapeDtypeStruct((NC, N, H), jnp.float32),
        jax.ShapeDtypeStruct((NC, N, H), jnp.float32),
    ),
    mesh=_sc_mesh,
    scratch_types=[
        pltpu.VMEM_SHARED((N, H), jnp.float32),
        pltpu.VMEM((3, L, H), jnp.float32),
        pltpu.VMEM((IB, L), jnp.int32),
        pltpu.SemaphoreType.DMA((3,)),
        pltpu.SemaphoreType.DMA((3,)),
    ],
)
def _sc_segment_sums(e_hbm, ie_hbm, eh_hbm, ih_hbm, zeros_hbm,
                     pe_out, ph_out, accum_sh, data_v, idx_v, sem_g, sem_s):
    c = lax.axis_index("c")
    s = lax.axis_index("s")
    wid = s * NC + c
    myrow = s * ROWS_A
    last = s == NS - 1

    def phase(src_hbm, idx_hbm, qb, rb, tail, out_ref, zero_first):
        gbase = (wid * qb + jnp.minimum(wid, rb)) * IB
        cnt = (qb + (wid < rb).astype(jnp.int32)) * IB
        cnt = cnt + jnp.where(wid == NW - 1, tail, 0)

        NBUF, D = 3, 2

        def wait_gather(b):
            pltpu.make_async_copy(src_hbm.at[pl.ds(0, L)], data_v.at[b],
                                  sem_g.at[b]).wait()

        def wait_scatter(b):
            pltpu.make_async_copy(data_v.at[b],
                                  accum_sh.at[idx_v.at[0]],
                                  sem_s.at[b]).wait()

        def start_gather(g, b):
            pltpu.async_copy(src_hbm.at[pl.ds((gbase + g) * L, L)],
                             data_v.at[b], sem_g.at[b])

        for g in range(D):
            start_gather(g, g % NBUF)

        if zero_first:
            pltpu.sync_copy(zeros_hbm, accum_sh.at[pl.ds(myrow, ROWS_A)])

            @pl.when(last)
            def _():
                pltpu.sync_copy(zeros_hbm.at[pl.ds(0, ROWS_TAIL)],
                                accum_sh.at[pl.ds(NS * ROWS_A, ROWS_TAIL)])

        plsc.subcore_barrier()

        def block(blk, carry):
            pltpu.sync_copy(idx_hbm.at[1, pl.ds(gbase + blk * IB, IB)], idx_v)
            for k in range(IB):
                sj = blk * IB + k
                gi = sj + D
                b = lax.rem(gi, NBUF)

                @pl.when(gi < cnt)
                def _():
                    @pl.when(gi >= NBUF)
                    def _():
                        wait_scatter(b)
                    start_gather(gi, b)

                bj = lax.rem(sj, NBUF)

                @pl.when(sj < cnt)
                def _():
                    wait_gather(bj)
                    pltpu.async_copy(data_v.at[bj],
                                     accum_sh.at[idx_v.at[k]],
                                     sem_s.at[bj], add=True)

            return carry

        lax.fori_loop(0, (cnt + IB - 1) // IB, block, 0)
        for b in range(NBUF):
            wait_scatter(b)
        plsc.subcore_barrier()
        pltpu.sync_copy(accum_sh.at[pl.ds(myrow, ROWS_A)],
                        out_ref.at[c, pl.ds(myrow, ROWS_A)])

        @pl.when(last)
        def _():
            pltpu.sync_copy(accum_sh.at[pl.ds(NS * ROWS_A, ROWS_TAIL)],
                            out_ref.at[c, pl.ds(NS * ROWS_A, ROWS_TAIL)])

    phase(e_hbm, ie_hbm, QBE, RBE, TAIL_E, pe_out, zero_first=True)
    phase(eh_hbm, ih_hbm, QBH, RBH, TAIL_H, ph_out, zero_first=False)


def _act(x):
    return jnp.maximum(x, 0.0) + jnp.log1p(jnp.exp(-jnp.abs(x))) - SHIFT


def _matT(x, w_ref):
    return lax.dot_general(x, w_ref[...], (((1,), (1,)), ((), ())),
                           preferred_element_type=jnp.float32)


def _mlp_body(pe_ref, ph_ref, v_ref, w1_ref, b1_ref, w2_ref, b2_ref,
              w1h_ref, b1h_ref, w2h_ref, b2h_ref, wc1_ref, wc2_ref, bc_ref,
              out_ref):
    a = pe_ref[0] + pe_ref[1]
    ah = (ph_ref[0] + ph_ref[1]) - a
    h = _matT(_act(_matT(a, w1_ref) + b1_ref[...]), w2_ref) + b2_ref[...]
    hh = _matT(_act(_matT(ah, w1h_ref) + b1h_ref[...]), w2h_ref) + b2h_ref[...]
    o = _act(_matT(h, wc1_ref) + _matT(hh, wc2_ref) + bc_ref[...])
    out_ref[...] = v_ref[...] + o


_MLP_B = 2000


def _mlp_call(pe, ph, v, *weights):
    part_spec = pl.BlockSpec((NC, _MLP_B, H), lambda i: (0, i, 0))
    row_spec = pl.BlockSpec((_MLP_B, H), lambda i: (i, 0))
    w_spec = pl.BlockSpec((H, H), lambda i: (0, 0))
    b_spec = pl.BlockSpec((1, H), lambda i: (0, 0))
    w_specs = [w_spec, b_spec] * 4 + [w_spec, w_spec, b_spec]
    return pl.pallas_call(
        _mlp_body,
        grid=(N // _MLP_B,),
        in_specs=[part_spec, part_spec, row_spec] + w_specs,
        out_specs=row_spec,
        out_shape=jax.ShapeDtypeStruct((N, H), jnp.float32),
    )(pe, ph, v, *weights)


def kernel(v, e, edge_index, e_hull, edge_index_hull,
           W1, b1, W2, b2, W1h, b1h, W2h, b2h, Wc, bc):
    ie = jnp.pad(edge_index.reshape(2, GE, L), ((0, 0), (0, PAD_E - GE), (0, 0)))
    ih = jnp.pad(edge_index_hull.reshape(2, GH, L), ((0, 0), (0, PAD_H - GH), (0, 0)))
    zeros = jnp.zeros((ROWS_A, H), jnp.float32)
    pe, ph = _sc_segment_sums(e, ie, e_hull, ih, zeros)
    return _mlp_call(
        pe, ph, v,
        W1, b1[None], W2, b2[None],
        W1h, b1h[None], W2h, b2h[None],
        Wc[:, :H], Wc[:, H:], bc[None])
